# Initial kernel scaffold; baseline (speedup 1.0000x reference)
import jax
import jax.numpy as jnp
from jax.experimental import pallas as pl


def kernel(x, edge_index, W_edge, W_self, b):
    raise NotImplementedError("write your pallas kernel here")


# speedup vs baseline: 5.9205x; 5.9205x over previous
import functools

import jax
import jax.numpy as jnp
from jax import lax
from jax.experimental import pallas as pl
from jax.experimental.pallas import tpu as pltpu
from jax.experimental.pallas import tpu_sc as plsc

N_NODES = 10000
N_EDGES = 320000
D = 128

CHUNK = 128
N_CHUNKS = N_EDGES // CHUNK
NC = 2
NS = 16
NW = NC * NS
NODES_PAD = 10112
ROWS_PER_TILE = NODES_PAD // NS

BASE_CHUNKS = N_CHUNKS // NW
EXTRA = N_CHUNKS - BASE_CHUNKS * NW


def _sc_segment_sum(x, src, dst, zrow, zdeg, ones):
    mesh = plsc.VectorSubcoreMesh(core_axis_name="c", subcore_axis_name="s")

    @functools.partial(
        pl.kernel,
        mesh=mesh,
        compiler_params=pltpu.CompilerParams(use_tc_tiling_on_sc=False),
        out_type=[
            jax.ShapeDtypeStruct((NC, NODES_PAD, D), jnp.float32),
            jax.ShapeDtypeStruct((NC, NODES_PAD, 16), jnp.float32),
        ],
        scratch_types=[
            pltpu.VMEM((CHUNK,), jnp.int32),
            pltpu.VMEM((1, CHUNK), jnp.int32),
            pltpu.VMEM((CHUNK, D), jnp.float32),
            pltpu.VMEM((CHUNK, 16), jnp.float32),
            pltpu.VMEM_SHARED((NODES_PAD, D), jnp.float32),
            pltpu.VMEM_SHARED((NODES_PAD, 16), jnp.float32),
            pltpu.SemaphoreType.DMA,
        ],
    )
    def seg_kernel(x_hbm, src_hbm, dst_hbm, zrow_hbm, zdeg_hbm, ones_hbm,
                   pooled_hbm, deg_hbm,
                   src_v, dst_v, rows_v, ones_v,
                   accum_sh, deg_sh, sem):
        cid = lax.axis_index("c")
        sid = lax.axis_index("s")
        wid = sid * NC + cid

        r0 = sid * ROWS_PER_TILE
        offs = [0, 128, 256, 384, 512]
        lens = [128, 128, 128, 128, ROWS_PER_TILE - 512]
        pltpu.sync_copy(zrow_hbm, rows_v)
        pltpu.sync_copy(zdeg_hbm, ones_v)
        for off, ln in zip(offs, lens):
            pltpu.sync_copy(rows_v.at[pl.ds(0, ln)],
                            accum_sh.at[pl.ds(r0 + off, ln)])
            pltpu.sync_copy(ones_v.at[pl.ds(0, ln)],
                            deg_sh.at[pl.ds(r0 + off, ln)])
        pltpu.sync_copy(ones_hbm, ones_v)
        plsc.subcore_barrier()

        n_chunks = BASE_CHUNKS + jnp.where(wid < EXTRA, 1, 0)
        chunk0 = wid * BASE_CHUNKS + jnp.minimum(wid, EXTRA)

        def body(j, carry):
            e0 = (chunk0 + j) * CHUNK
            pltpu.sync_copy(src_hbm.at[pl.ds(e0, CHUNK)], src_v)
            pltpu.sync_copy(dst_hbm.at[pl.ds(e0, CHUNK)], dst_v.at[0])
            pltpu.async_copy(x_hbm.at[src_v], rows_v, sem).wait()
            pltpu.sync_copy(rows_v, accum_sh.at[dst_v.at[0]], add=True)
            pltpu.sync_copy(ones_v, deg_sh.at[dst_v.at[0]], add=True)
            return carry

        lax.fori_loop(0, n_chunks, body, 0)
        plsc.subcore_barrier()

        for off, ln in zip(offs, lens):
            rr = r0 + off
            pltpu.sync_copy(accum_sh.at[pl.ds(rr, ln)], rows_v.at[pl.ds(0, ln)])
            pltpu.sync_copy(rows_v.at[pl.ds(0, ln)],
                            pooled_hbm.at[cid, pl.ds(rr, ln)])
            pltpu.sync_copy(deg_sh.at[pl.ds(rr, ln)], ones_v.at[pl.ds(0, ln)])
            pltpu.sync_copy(ones_v.at[pl.ds(0, ln)],
                            deg_hbm.at[cid, pl.ds(rr, ln)])

    return seg_kernel(x, src, dst, zrow, zdeg, ones)


def _tc_combine_body(p0, p1, d0, d1, x, we, ws, b, out):
    pooled = p0[...] + p1[...]
    e = jnp.dot(pooled, we[...], preferred_element_type=jnp.float32)
    s = jnp.dot(x[...], ws[...], preferred_element_type=jnp.float32)
    denom = d0[:, 0:1] + d1[:, 0:1] + 1.0
    out[...] = jnp.maximum((e + s) / denom + b[...], 0.0)


def kernel(x, edge_index, W_edge, W_self, b):
    src = edge_index[0]
    dst = edge_index[1]
    zrow = jnp.zeros((CHUNK, D), jnp.float32)
    zdeg = jnp.zeros((CHUNK, 16), jnp.float32)
    ones = jnp.ones((CHUNK, 16), jnp.float32)

    pooled, deg = _sc_segment_sum(x, src, dst, zrow, zdeg, ones)

    blk = 256
    grid = (N_NODES + blk - 1) // blk
    out = pl.pallas_call(
        _tc_combine_body,
        grid=(grid,),
        in_specs=[
            pl.BlockSpec((blk, D), lambda i: (i, 0)),
            pl.BlockSpec((blk, D), lambda i: (i, 0)),
            pl.BlockSpec((blk, 16), lambda i: (i, 0)),
            pl.BlockSpec((blk, 16), lambda i: (i, 0)),
            pl.BlockSpec((blk, D), lambda i: (i, 0)),
            pl.BlockSpec((D, D), lambda i: (0, 0)),
            pl.BlockSpec((D, D), lambda i: (0, 0)),
            pl.BlockSpec((1, D), lambda i: (0, 0)),
        ],
        out_specs=pl.BlockSpec((blk, D), lambda i: (i, 0)),
        out_shape=jax.ShapeDtypeStruct((N_NODES, D), jnp.float32),
    )(pooled[0], pooled[1], deg[0], deg[1], x, W_edge, W_self, b.reshape(1, D))
    return out
